# baseline (speedup 1.0000x reference)
import jax
import jax.numpy as jnp
from jax import lax
from jax.experimental import pallas as pl
from jax.experimental.pallas import tpu as pltpu
from jax.experimental.pallas import tpu_sc as plsc

BATCH = 16384
PAIR = 128
IDX_MINOR = 128
L = 16
NW = 32
BPW = BATCH // NW


def _dot_pass(t2, idxs, odds, rows, wv, o0v, o1v, sem, wcol, init0, init1,
              first):
    n_chunks = BPW // IDX_MINOR
    copies = []
    for k in range(n_chunks):
        copies.append(pltpu.async_copy(
            t2.at[idxs.at[k]], rows.at[pl.ds(IDX_MINOR * k, IDX_MINOR)], sem))
    for c in copies:
        c.wait()

    w0 = [wv[0, pl.ds(wcol + L * k, L)] for k in range(4)]
    w1 = [wv[1, pl.ds(wcol + L * k, L)] for k in range(4)]
    lastmask = lax.iota(jnp.int32, L) == (L - 1)

    def body(b, carry):
        oddv = odds[pl.ds(b * L, L)]
        for r in range(L):
            i = b * L + r
            rl = [rows[i, pl.ds(L * k, L)] for k in range(8)]
            a0lo, a0hi = init0, init0
            a1lo, a1hi = init1, init1
            for k in range(4):
                a0lo = a0lo + rl[k] * w0[k]
                a0hi = a0hi + rl[4 + k] * w0[k]
                a1lo = a1lo + rl[k] * w1[k]
                a1hi = a1hi + rl[4 + k] * w1[k]
            om = oddv.at[jnp.full((L,), r, jnp.int32)].get(
                mode="promise_in_bounds") == 1
            s0 = plsc.cumsum(jnp.where(om, a0hi, a0lo))
            s1 = plsc.cumsum(jnp.where(om, a1hi, a1lo))
            idxv = jnp.zeros((L,), jnp.int32) + i
            if first:
                plsc.store_scatter(o0v, [idxv], s0, mask=lastmask)
                plsc.store_scatter(o1v, [idxv], s1, mask=lastmask)
            else:
                plsc.addupdate_scatter(o0v, [idxv], s0, mask=lastmask)
                plsc.addupdate_scatter(o1v, [idxv], s1, mask=lastmask)
        return carry

    lax.fori_loop(0, BPW // L, body, 0)


def _sc_kernel(t2, hs2, ts2, ho, to, w_t, b16, out0, out1,
               hsv, tsv, hov, tov, rows, wv, bv, o0v, o1v, sem):
    nc = 2
    wid = lax.axis_index("s") * nc + lax.axis_index("c")
    n_chunks = BPW // IDX_MINOR

    pltpu.sync_copy(hs2.at[pl.ds(wid * n_chunks, n_chunks)], hsv)
    pltpu.sync_copy(ts2.at[pl.ds(wid * n_chunks, n_chunks)], tsv)
    pltpu.sync_copy(ho.at[pl.ds(wid * BPW, BPW)], hov)
    pltpu.sync_copy(to.at[pl.ds(wid * BPW, BPW)], tov)
    pltpu.sync_copy(w_t, wv)
    pltpu.sync_copy(b16, bv)

    binit0 = bv[0, pl.ds(0, L)]
    binit1 = bv[1, pl.ds(0, L)]
    zinit = jnp.zeros((L,), jnp.float32)

    _dot_pass(t2, hsv, hov, rows, wv, o0v, o1v, sem, 0, binit0, binit1, True)
    _dot_pass(t2, tsv, tov, rows, wv, o0v, o1v, sem, 64, zinit, zinit, False)

    pltpu.sync_copy(o0v, out0.at[pl.ds(wid * BPW, BPW)])
    pltpu.sync_copy(o1v, out1.at[pl.ds(wid * BPW, BPW)])


def kernel(head, tail, table, fc_w, fc_b):
    t2 = table.reshape(BATCH * 0 + 500000, PAIR)
    hs2 = (head >> 1).reshape(BATCH // IDX_MINOR, IDX_MINOR)
    ts2 = (tail >> 1).reshape(BATCH // IDX_MINOR, IDX_MINOR)
    ho = head & 1
    to = tail & 1
    w_t = fc_w.T
    b16 = jnp.zeros((2, L), jnp.float32).at[:, 0].set(fc_b)

    mesh = plsc.VectorSubcoreMesh(core_axis_name="c", subcore_axis_name="s")
    n_chunks = BPW // IDX_MINOR
    run = pl.kernel(
        _sc_kernel,
        mesh=mesh,
        compiler_params=pltpu.CompilerParams(needs_layout_passes=False),
        out_type=[
            jax.ShapeDtypeStruct((BATCH,), jnp.float32),
            jax.ShapeDtypeStruct((BATCH,), jnp.float32),
        ],
        scratch_types=[
            pltpu.VMEM((n_chunks, IDX_MINOR), jnp.int32),
            pltpu.VMEM((n_chunks, IDX_MINOR), jnp.int32),
            pltpu.VMEM((BPW,), jnp.int32),
            pltpu.VMEM((BPW,), jnp.int32),
            pltpu.VMEM((BPW, PAIR), jnp.float32),
            pltpu.VMEM((2, 2 * 64), jnp.float32),
            pltpu.VMEM((2, L), jnp.float32),
            pltpu.VMEM((BPW,), jnp.float32),
            pltpu.VMEM((BPW,), jnp.float32),
            pltpu.SemaphoreType.DMA,
        ],
    )
    o0, o1 = run(t2, hs2, ts2, ho, to, w_t, b16)
    return jnp.stack([o0, o1], axis=1)


# speedup vs baseline: 4.0179x; 4.0179x over previous
import jax
import jax.numpy as jnp
from jax import lax
from jax.experimental import pallas as pl
from jax.experimental.pallas import tpu as pltpu
from jax.experimental.pallas import tpu_sc as plsc

VOCAB = 1000000
EMBED = 64
BATCH = 16384
IDX_MINOR = 128
L = 16
NW = 32
BPW = BATCH // NW
NCH = BPW // IDX_MINOR
CHUNK = 8192


def _mm_kernel(t_blk, w_blk, p0, p1, p2, p3):
    x = t_blk[...]
    w = w_blk[...]
    y = lax.dot_general(w, x, (((0,), (0,)), ((), ())),
                        preferred_element_type=jnp.float32)
    p0[...] = y[0, :]
    p1[...] = y[1, :]
    p2[...] = y[2, :]
    p3[...] = y[3, :]


def _sc_kernel(p0, p1, p2, p3, h2, t2, bsp, out0, out1,
               hidx, tidx, g0, g1, g2, g3, bv, o0v, o1v, sem):
    wid = lax.axis_index("s") * 2 + lax.axis_index("c")

    pltpu.sync_copy(h2.at[pl.ds(wid * NCH, NCH)], hidx)
    pltpu.sync_copy(t2.at[pl.ds(wid * NCH, NCH)], tidx)
    pltpu.sync_copy(bsp, bv)

    copies = []
    for k in range(NCH):
        sl = pl.ds(IDX_MINOR * k, IDX_MINOR)
        copies.append(pltpu.async_copy(p0.at[hidx.at[k]], g0.at[sl], sem))
        copies.append(pltpu.async_copy(p1.at[hidx.at[k]], g1.at[sl], sem))
        copies.append(pltpu.async_copy(p2.at[tidx.at[k]], g2.at[sl], sem))
        copies.append(pltpu.async_copy(p3.at[tidx.at[k]], g3.at[sl], sem))
    for c in copies:
        c.wait()

    b0 = bv[0, pl.ds(0, L)]
    b1 = bv[1, pl.ds(0, L)]

    def body(sv, carry):
        sl = pl.ds(sv * L, L)
        o0v[sl] = g0[sl] + g2[sl] + b0
        o1v[sl] = g1[sl] + g3[sl] + b1
        return carry

    lax.fori_loop(0, BPW // L, body, 0)

    pltpu.sync_copy(o0v, out0.at[pl.ds(wid * BPW, BPW)])
    pltpu.sync_copy(o1v, out1.at[pl.ds(wid * BPW, BPW)])


def kernel(head, tail, table, fc_w, fc_b):
    tT = table.T
    w8 = jnp.zeros((EMBED, 8), jnp.float32)
    w8 = w8.at[:, 0].set(fc_w[:EMBED, 0]).at[:, 1].set(fc_w[:EMBED, 1])
    w8 = w8.at[:, 2].set(fc_w[EMBED:, 0]).at[:, 3].set(fc_w[EMBED:, 1])

    grid = pl.cdiv(VOCAB, CHUNK)
    planes = pl.pallas_call(
        _mm_kernel,
        grid=(grid,),
        in_specs=[
            pl.BlockSpec((EMBED, CHUNK), lambda i: (0, i)),
            pl.BlockSpec((EMBED, 8), lambda i: (0, 0)),
        ],
        out_specs=[pl.BlockSpec((CHUNK,), lambda i: (i,))] * 4,
        out_shape=[jax.ShapeDtypeStruct((VOCAB,), jnp.float32)] * 4,
    )(tT, w8)
    p0, p1, p2, p3 = planes

    h2 = head.reshape(BATCH // IDX_MINOR, IDX_MINOR)
    t2 = tail.reshape(BATCH // IDX_MINOR, IDX_MINOR)
    bsp = jnp.broadcast_to(fc_b[:, None], (2, L))

    mesh = plsc.VectorSubcoreMesh(core_axis_name="c", subcore_axis_name="s")
    run = pl.kernel(
        _sc_kernel,
        mesh=mesh,
        compiler_params=pltpu.CompilerParams(
            needs_layout_passes=False, use_tc_tiling_on_sc=False),
        out_type=[
            jax.ShapeDtypeStruct((BATCH,), jnp.float32),
            jax.ShapeDtypeStruct((BATCH,), jnp.float32),
        ],
        scratch_types=[
            pltpu.VMEM((NCH, IDX_MINOR), jnp.int32),
            pltpu.VMEM((NCH, IDX_MINOR), jnp.int32),
            pltpu.VMEM((BPW,), jnp.float32),
            pltpu.VMEM((BPW,), jnp.float32),
            pltpu.VMEM((BPW,), jnp.float32),
            pltpu.VMEM((BPW,), jnp.float32),
            pltpu.VMEM((2, L), jnp.float32),
            pltpu.VMEM((BPW,), jnp.float32),
            pltpu.VMEM((BPW,), jnp.float32),
            pltpu.SemaphoreType.DMA,
        ],
    )
    o0, o1 = run(p0, p1, p2, p3, h2, t2, bsp)
    return jnp.stack([o0, o1], axis=1)


# speedup vs baseline: 5.6723x; 1.4118x over previous
import jax
import jax.numpy as jnp
from jax import lax
from jax.experimental import pallas as pl
from jax.experimental.pallas import tpu as pltpu
from jax.experimental.pallas import tpu_sc as plsc

VOCAB = 1000000
EMBED = 64
BATCH = 16384
IDX_MINOR = 128
L = 16
NW = 32
BPW = BATCH // NW
NCH = BPW // IDX_MINOR
CHUNK = 32768


def _mm_kernel(t_blk, w_blk, p0, p1, p2, p3):
    x = t_blk[...]
    w = w_blk[...]
    y = lax.dot_general(w, x, (((0,), (0,)), ((), ())),
                        preferred_element_type=jnp.float32)
    p0[...] = y[0, :]
    p1[...] = y[1, :]
    p2[...] = y[2, :]
    p3[...] = y[3, :]


def _sc_kernel(p0, p1, p2, p3, h2, t2, bsp, out0, out1,
               hidx, tidx, g0, g1, g2, g3, bv, o0v, o1v, sem):
    wid = lax.axis_index("s") * 2 + lax.axis_index("c")

    pltpu.sync_copy(h2.at[pl.ds(wid * NCH, NCH)], hidx)
    pltpu.sync_copy(t2.at[pl.ds(wid * NCH, NCH)], tidx)
    pltpu.sync_copy(bsp, bv)

    copies = []
    for k in range(NCH):
        sl = pl.ds(IDX_MINOR * k, IDX_MINOR)
        copies.append(pltpu.async_copy(p0.at[hidx.at[k]], g0.at[sl], sem))
        copies.append(pltpu.async_copy(p1.at[hidx.at[k]], g1.at[sl], sem))
        copies.append(pltpu.async_copy(p2.at[tidx.at[k]], g2.at[sl], sem))
        copies.append(pltpu.async_copy(p3.at[tidx.at[k]], g3.at[sl], sem))
    for c in copies:
        c.wait()

    b0 = bv[0, pl.ds(0, L)]
    b1 = bv[1, pl.ds(0, L)]

    def body(sv, carry):
        sl = pl.ds(sv * L, L)
        o0v[sl] = g0[sl] + g2[sl] + b0
        o1v[sl] = g1[sl] + g3[sl] + b1
        return carry

    lax.fori_loop(0, BPW // L, body, 0)

    pltpu.sync_copy(o0v, out0.at[pl.ds(wid * BPW, BPW)])
    pltpu.sync_copy(o1v, out1.at[pl.ds(wid * BPW, BPW)])


def kernel(head, tail, table, fc_w, fc_b):
    tT = table.T
    w8 = jnp.zeros((EMBED, 8), jnp.float32)
    w8 = w8.at[:, 0].set(fc_w[:EMBED, 0]).at[:, 1].set(fc_w[:EMBED, 1])
    w8 = w8.at[:, 2].set(fc_w[EMBED:, 0]).at[:, 3].set(fc_w[EMBED:, 1])

    grid = pl.cdiv(VOCAB, CHUNK)
    planes = pl.pallas_call(
        _mm_kernel,
        grid=(grid,),
        in_specs=[
            pl.BlockSpec((EMBED, CHUNK), lambda i: (0, i)),
            pl.BlockSpec((EMBED, 8), lambda i: (0, 0)),
        ],
        out_specs=[pl.BlockSpec((CHUNK,), lambda i: (i,))] * 4,
        out_shape=[jax.ShapeDtypeStruct((VOCAB,), jnp.float32)] * 4,
    )(tT, w8)
    p0, p1, p2, p3 = planes

    h2 = head.reshape(BATCH // IDX_MINOR, IDX_MINOR)
    t2 = tail.reshape(BATCH // IDX_MINOR, IDX_MINOR)
    bsp = jnp.broadcast_to(fc_b[:, None], (2, L))

    mesh = plsc.VectorSubcoreMesh(core_axis_name="c", subcore_axis_name="s")
    run = pl.kernel(
        _sc_kernel,
        mesh=mesh,
        compiler_params=pltpu.CompilerParams(
            needs_layout_passes=False, use_tc_tiling_on_sc=False),
        out_type=[
            jax.ShapeDtypeStruct((BATCH,), jnp.float32),
            jax.ShapeDtypeStruct((BATCH,), jnp.float32),
        ],
        scratch_types=[
            pltpu.VMEM((NCH, IDX_MINOR), jnp.int32),
            pltpu.VMEM((NCH, IDX_MINOR), jnp.int32),
            pltpu.VMEM((BPW,), jnp.float32),
            pltpu.VMEM((BPW,), jnp.float32),
            pltpu.VMEM((BPW,), jnp.float32),
            pltpu.VMEM((BPW,), jnp.float32),
            pltpu.VMEM((2, L), jnp.float32),
            pltpu.VMEM((BPW,), jnp.float32),
            pltpu.VMEM((BPW,), jnp.float32),
            pltpu.SemaphoreType.DMA,
        ],
    )
    o0, o1 = run(p0, p1, p2, p3, h2, t2, bsp)
    return jnp.stack([o0, o1], axis=1)
